# baseline (speedup 1.0000x reference)
import jax
import jax.numpy as jnp
from jax import lax
from jax.experimental import pallas as pl
from jax.experimental.pallas import tpu as pltpu
from jax.experimental.pallas import tpu_sc as plsc

_VOCAB, _HIDDEN, _NSPK, _B, _L = 1000, 384, 10, 64, 200
_NC, _NS = 2, 16
_NW = _NC * _NS
_BPW = _B // _NW
_CHUNKS = ((0, 104), (104, 96))
_KL = _HIDDEN // 16


def _speaker_tc_body(ids_ref, table_ref, w_ref, b_ref, out_ref):
    ids = ids_ref[:]
    onehot = (lax.broadcasted_iota(jnp.int32, (_B, _NSPK), 1) == ids)
    emb = jnp.dot(onehot.astype(jnp.float32), table_ref[:],
                  preferred_element_type=jnp.float32)
    x = jnp.dot(emb, w_ref[:], preferred_element_type=jnp.float32) + b_ref[:]
    out_ref[:] = jnp.maximum(x, 0.0) + jnp.log1p(jnp.exp(-jnp.abs(x)))


def _speaker_features(speaker_ids, speaker_table, fc_w, fc_b):
    return pl.pallas_call(
        _speaker_tc_body,
        out_shape=jax.ShapeDtypeStruct((_B, _HIDDEN), jnp.float32),
    )(speaker_ids.reshape(_B, 1), speaker_table, fc_w, fc_b.reshape(1, _HIDDEN))


def _sc_body(ids_hbm, char_hbm, pos_hbm, spk_hbm, out_hbm,
             idx, spk_row, pos_res, buf, sem):
    wid = lax.axis_index("s") * _NC + lax.axis_index("c")
    pltpu.sync_copy(pos_hbm, pos_res)
    for j in range(_BPW):
        b = wid * _BPW + j
        pltpu.sync_copy(ids_hbm.at[pl.ds(b * _L, _L)], idx)
        pltpu.sync_copy(spk_hbm.at[pl.ds(b * _HIDDEN, _HIDDEN)], spk_row)
        spk_vecs = [spk_row[pl.ds(k * 16, 16)] for k in range(_KL)]
        for c0, n in _CHUNKS:
            pltpu.async_copy(char_hbm.at[idx.at[pl.ds(c0, n)]],
                             buf.at[pl.ds(0, n)], sem).wait()

            def row(i, _):
                for k in range(_KL):
                    sl = pl.ds(k * 16, 16)
                    buf[i, sl] = buf[i, sl] + pos_res[c0 + i, sl] + spk_vecs[k]
                return 0

            lax.fori_loop(0, n, row, 0)
            pltpu.sync_copy(buf.at[pl.ds(0, n)], out_hbm.at[b, pl.ds(c0, n)])


def kernel(input_ids, speaker_ids, charactor_embeddings, position_table,
           speaker_table, fc_w, fc_b):
    spk_feat = _speaker_features(speaker_ids, speaker_table, fc_w, fc_b)
    mesh = plsc.VectorSubcoreMesh(core_axis_name="c", subcore_axis_name="s")
    run = pl.kernel(
        _sc_body,
        out_type=jax.ShapeDtypeStruct((_B, _L, _HIDDEN), jnp.float32),
        mesh=mesh,
        scratch_types=[
            pltpu.VMEM((_L,), jnp.int32),
            pltpu.VMEM((_HIDDEN,), jnp.float32),
            pltpu.VMEM((_L, _HIDDEN), jnp.float32),
            pltpu.VMEM((104, _HIDDEN), jnp.float32),
            pltpu.SemaphoreType.DMA,
        ],
    )
    return run(input_ids.reshape(-1), charactor_embeddings,
               position_table[1:_L + 1], spk_feat.reshape(-1))
